# Initial kernel scaffold; baseline (speedup 1.0000x reference)
import jax
import jax.numpy as jnp
from jax.experimental import pallas as pl


def kernel(hidden_states, gate_w, w_gate_up, w_down):
    raise NotImplementedError("write your pallas kernel here")


# speedup vs baseline: 2.6891x; 2.6891x over previous
import functools

import jax
import jax.numpy as jnp
from jax import lax
from jax.experimental import pallas as pl
from jax.experimental.pallas import tpu as pltpu
from jax.experimental.pallas import tpu_sc as plsc

B, S, H = 2, 2048, 2048
DFF = 2048
E = 8
N = B * S
NS = 2 * N
BLK = 256
RB = NS // BLK
T = RB + E
DB = 2
DBLK = DFF // DB

NW = 32
SLOTS_W = NS // NW
CH = 16
NCH = SLOTS_W // CH

@functools.cache
def _sc_mesh():
    return plsc.VectorSubcoreMesh(core_axis_name="c", subcore_axis_name="s")


def _gelu(x):
    return 0.5 * x * (1.0 + lax.erf(x * 0.7071067811865476))


def _logits_body(x_ref, gw_ref, o_ref):
    o_ref[...] = lax.dot_general(
        x_ref[...].astype(jnp.bfloat16), gw_ref[...].astype(jnp.bfloat16),
        (((1,), (1,)), ((), ())),
        preferred_element_type=jnp.float32)


def _router_logits(x, gate_w):
    return pl.pallas_call(
        _logits_body,
        grid=(8,),
        in_specs=[
            pl.BlockSpec((N // 8, H), lambda i: (i, 0)),
            pl.BlockSpec((E, H), lambda i: (0, 0)),
        ],
        out_specs=pl.BlockSpec((N // 8, E), lambda i: (i, 0)),
        out_shape=jax.ShapeDtypeStruct((N, E), jnp.float32),
    )(x, gate_w)


def _route_body(l_ref, p_ref, w_ref, offs_ref):
    logits = l_ref[...]
    m = jnp.max(logits, axis=1, keepdims=True)
    ex = jnp.exp(logits - m)
    s = jnp.sum(ex, axis=1, keepdims=True)
    probs = ex / s
    ii = lax.broadcasted_iota(jnp.int32, (N, E), 1)
    p1 = jnp.max(probs, axis=1, keepdims=True)
    a1 = jnp.min(jnp.where(probs == p1, ii, E), axis=1, keepdims=True)
    sel1 = ii == a1
    pm = jnp.where(sel1, -1.0, probs)
    p2 = jnp.max(pm, axis=1, keepdims=True)
    a2 = jnp.min(jnp.where(pm == p2, ii, E), axis=1, keepdims=True)
    denom = p1 + p2
    w1 = p1 / denom
    w2 = p2 / denom

    onehot = jnp.concatenate([sel1.astype(jnp.float32),
                              (ii == a2).astype(jnp.float32)], axis=0)

    li = lax.broadcasted_iota(jnp.int32, (512, 512), 0)
    lj = lax.broadcasted_iota(jnp.int32, (512, 512), 1)
    ltri = (li >= lj).astype(jnp.float32)

    carry = jnp.zeros((1, E), jnp.float32)
    cblocks = []
    for i in range(NS // 512):
        blk = lax.slice(onehot, (i * 512, 0), ((i + 1) * 512, E))
        c = lax.dot_general(ltri, blk, (((1,), (0,)), ((), ())),
                            preferred_element_type=jnp.float32) + carry
        cblocks.append(c)
        carry = carry + jnp.sum(blk, axis=0, keepdims=True)
    csum = jnp.concatenate(cblocks, axis=0)
    counts = carry
    mi = lax.broadcasted_iota(jnp.int32, (E, E), 0)
    mj = lax.broadcasted_iota(jnp.int32, (E, E), 1)
    mstrict = (mi < mj).astype(jnp.float32)
    offs = lax.dot_general(counts, mstrict, (((1,), (0,)), ((), ())),
                           preferred_element_type=jnp.float32,
                           precision=lax.Precision.HIGHEST)

    pos = jnp.sum(onehot * (csum - 1.0 + offs), axis=1,
                  keepdims=True)
    p_ref[...] = jnp.broadcast_to(pos, (NS, E))
    wcat = jnp.concatenate([w1, w2], axis=0)
    w_ref[...] = jnp.broadcast_to(wcat, (NS, E))
    offs_ref[...] = jnp.broadcast_to(offs, (E, E))


def _route(logits):
    return pl.pallas_call(
        _route_body,
        out_shape=[
            jax.ShapeDtypeStruct((NS, E), jnp.float32),
            jax.ShapeDtypeStruct((NS, E), jnp.float32),
            jax.ShapeDtypeStruct((E, E), jnp.float32),
        ],
    )(logits)


def _tile_maps(offs):
    starts = offs
    ends = jnp.concatenate([offs[1:], jnp.array([NS], jnp.int32)])
    rb = jnp.arange(RB, dtype=jnp.int32)
    lo = rb * BLK
    hi = lo + BLK
    e_lo = jnp.searchsorted(ends, lo, side="right").astype(jnp.int32)
    e_hi = (jnp.searchsorted(starts, hi, side="left") - 1).astype(jnp.int32)
    n = e_hi - e_lo + 1
    cumstart = jnp.concatenate(
        [jnp.zeros((1,), jnp.int32), jnp.cumsum(n)[:-1].astype(jnp.int32)])
    t = jnp.arange(T, dtype=jnp.int32)
    rb_t = jnp.clip(jnp.searchsorted(cumstart, t, side="right").astype(
        jnp.int32) - 1, 0, RB - 1)
    within = t - cumstart[rb_t]
    valid = (within < n[rb_t]).astype(jnp.int32)
    e_t = jnp.clip(e_lo[rb_t] + within, 0, E - 1)
    first = ((within == 0) & (valid == 1)).astype(jnp.int32)
    return rb_t, e_t, first, valid, starts, ends


def _sc_scatter_rows(x, p3):
    @functools.partial(
        pl.kernel,
        out_type=jax.ShapeDtypeStruct((NS, H), jnp.float32),
        mesh=_sc_mesh(),
        scratch_types=[
            pltpu.VMEM((NCH, CH), jnp.int32),
            pltpu.VMEM((CH, H), jnp.float32),
            pltpu.SemaphoreType.DMA,
        ],
    )
    def k(x_hbm, p_hbm, xs_hbm, idx_v, rows_v, sem):
        wid = lax.axis_index("s") * 2 + lax.axis_index("c")
        pltpu.sync_copy(p_hbm.at[wid], idx_v)
        base = wid * SLOTS_W
        for j in range(NCH):
            row0 = (base + j * CH) % N
            pltpu.sync_copy(x_hbm.at[pl.ds(row0, CH)], rows_v)
            pltpu.async_copy(rows_v, xs_hbm.at[idx_v.at[j]], sem).wait()

    return k(x, p3)


def _sc_gather_rows(ys, p3):
    @functools.partial(
        pl.kernel,
        out_type=jax.ShapeDtypeStruct((NS, H), jnp.float32),
        mesh=_sc_mesh(),
        scratch_types=[
            pltpu.VMEM((NCH, CH), jnp.int32),
            pltpu.VMEM((CH, H), jnp.float32),
            pltpu.SemaphoreType.DMA,
        ],
    )
    def k(ys_hbm, p_hbm, yo_hbm, idx_v, rows_v, sem):
        wid = lax.axis_index("s") * 2 + lax.axis_index("c")
        pltpu.sync_copy(p_hbm.at[wid], idx_v)
        base = wid * SLOTS_W
        for j in range(NCH):
            pltpu.async_copy(ys_hbm.at[idx_v.at[j]], rows_v, sem).wait()
            pltpu.sync_copy(rows_v, yo_hbm.at[pl.ds(base + j * CH, CH)])

    return k(ys, p3)


def _gu_body(rbt_r, et_r, first_r, valid_r, st_r, en_r,
             xs_ref, wg_ref, wu_ref, h_ref):
    t = pl.program_id(1)
    e = et_r[t]
    rows = rbt_r[t] * BLK + lax.broadcasted_iota(jnp.int32, (BLK, 1), 0)
    m = ((rows >= st_r[e]) & (rows < en_r[e])).astype(jnp.float32)
    m = m * valid_r[t].astype(jnp.float32)
    x = xs_ref[...]
    gate = lax.dot_general(x, wg_ref[0], (((1,), (1,)), ((), ())),
                           preferred_element_type=jnp.float32)
    up = lax.dot_general(x, wu_ref[0], (((1,), (1,)), ((), ())),
                         preferred_element_type=jnp.float32)
    h = up * _gelu(gate) * m

    @pl.when(first_r[t] == 1)
    def _():
        h_ref[...] = h

    @pl.when(first_r[t] == 0)
    def _():
        h_ref[...] += h


def _gmm_gateup(rbt, et, first, valid, st, en, xs, w_gate_up):
    grid_spec = pltpu.PrefetchScalarGridSpec(
        num_scalar_prefetch=6,
        grid=(DB, T),
        in_specs=[
            pl.BlockSpec((BLK, H), lambda d, t, rbt, *_: (rbt[t], 0)),
            pl.BlockSpec((1, DBLK, H),
                         lambda d, t, rbt, et, *_: (et[t], d, 0)),
            pl.BlockSpec((1, DBLK, H),
                         lambda d, t, rbt, et, *_: (et[t], d + DB, 0)),
        ],
        out_specs=pl.BlockSpec((BLK, DBLK), lambda d, t, rbt, *_: (rbt[t], d)),
    )
    return pl.pallas_call(
        _gu_body,
        grid_spec=grid_spec,
        out_shape=jax.ShapeDtypeStruct((NS, DFF), jnp.float32),
        compiler_params=pltpu.CompilerParams(
            dimension_semantics=("arbitrary", "arbitrary")),
    )(rbt, et, first, valid, st, en, xs, w_gate_up, w_gate_up)


def _down_body(rbt_r, et_r, first_r, valid_r, st_r, en_r,
               h_ref, wd_ref, y_ref):
    t = pl.program_id(0)
    e = et_r[t]
    rows = rbt_r[t] * BLK + lax.broadcasted_iota(jnp.int32, (BLK, 1), 0)
    m = ((rows >= st_r[e]) & (rows < en_r[e])).astype(jnp.float32)
    m = m * valid_r[t].astype(jnp.float32)
    y = lax.dot_general(h_ref[...] * m, wd_ref[0], (((1,), (1,)), ((), ())),
                        preferred_element_type=jnp.float32)

    @pl.when(first_r[t] == 1)
    def _():
        y_ref[...] = y

    @pl.when(first_r[t] == 0)
    def _():
        y_ref[...] += y


def _gmm_down(rbt, et, first, valid, st, en, h, w_down):
    grid_spec = pltpu.PrefetchScalarGridSpec(
        num_scalar_prefetch=6,
        grid=(T,),
        in_specs=[
            pl.BlockSpec((BLK, DFF), lambda t, rbt, *_: (rbt[t], 0)),
            pl.BlockSpec((1, H, DFF), lambda t, rbt, et, *_: (et[t], 0, 0)),
        ],
        out_specs=pl.BlockSpec((BLK, H), lambda t, rbt, *_: (rbt[t], 0)),
    )
    return pl.pallas_call(
        _down_body,
        grid_spec=grid_spec,
        out_shape=jax.ShapeDtypeStruct((NS, H), jnp.float32),
        compiler_params=pltpu.CompilerParams(
            dimension_semantics=("arbitrary",)),
    )(rbt, et, first, valid, st, en, h, w_down)


def _comb_body(y1_ref, y2_ref, w1_ref, w2_ref, o_ref):
    o_ref[...] = (y1_ref[...] * w1_ref[:, 0:1] +
                  y2_ref[...] * w2_ref[:, 0:1])


def _combine(y_slot, w_b):
    blk = N // 8
    return pl.pallas_call(
        _comb_body,
        grid=(8,),
        in_specs=[
            pl.BlockSpec((blk, H), lambda i: (i, 0)),
            pl.BlockSpec((blk, H), lambda i: (i + 8, 0)),
            pl.BlockSpec((blk, E), lambda i: (i, 0)),
            pl.BlockSpec((blk, E), lambda i: (i + 8, 0)),
        ],
        out_specs=pl.BlockSpec((blk, H), lambda i: (i, 0)),
        out_shape=jax.ShapeDtypeStruct((N, H), jnp.float32),
    )(y_slot, y_slot, w_b, w_b)


def kernel(hidden_states, gate_w, w_gate_up, w_down):
    x = hidden_states.reshape(N, H)
    logits = _router_logits(x, gate_w)
    p_b, w_b, offs_b = _route(logits)
    p = p_b[:, 0].astype(jnp.int32)
    offs = offs_b[0, :].astype(jnp.int32)
    p3 = p.reshape(NW, NCH, CH)
    rbt, et, first, valid, st, en = _tile_maps(offs)
    xs = _sc_scatter_rows(x, p3)
    h = _gmm_gateup(rbt, et, first, valid, st, en, xs, w_gate_up)
    ys = _gmm_down(rbt, et, first, valid, st, en, h, w_down)
    y_slot = _sc_gather_rows(ys, p3)
    out = _combine(y_slot, w_b)
    return out.reshape(B, S, H)


# speedup vs baseline: 2.7611x; 1.0268x over previous
import functools

import jax
import jax.numpy as jnp
from jax import lax
from jax.experimental import pallas as pl
from jax.experimental.pallas import tpu as pltpu
from jax.experimental.pallas import tpu_sc as plsc

B, S, H = 2, 2048, 2048
DFF = 2048
E = 8
N = B * S
NS = 2 * N
BLK = 256
RB = NS // BLK
T = RB + E
DB = 2
DBLK = DFF // DB

NW = 32
SLOTS_W = NS // NW
CH = 16
NCH = SLOTS_W // CH

@functools.cache
def _sc_mesh():
    return plsc.VectorSubcoreMesh(core_axis_name="c", subcore_axis_name="s")


def _gelu(x):
    return 0.5 * x * (1.0 + lax.erf(x * 0.7071067811865476))


def _logits_body(x_ref, gw_ref, o_ref):
    o_ref[...] = lax.dot_general(
        x_ref[...].astype(jnp.bfloat16), gw_ref[...].astype(jnp.bfloat16),
        (((1,), (1,)), ((), ())),
        preferred_element_type=jnp.float32)


def _router_logits(x, gate_w):
    return pl.pallas_call(
        _logits_body,
        grid=(8,),
        in_specs=[
            pl.BlockSpec((N // 8, H), lambda i: (i, 0)),
            pl.BlockSpec((E, H), lambda i: (0, 0)),
        ],
        out_specs=pl.BlockSpec((N // 8, E), lambda i: (i, 0)),
        out_shape=jax.ShapeDtypeStruct((N, E), jnp.float32),
    )(x, gate_w)


def _route_body(l_ref, p_ref, w_ref, offs_ref):
    logits = l_ref[...]
    m = jnp.max(logits, axis=1, keepdims=True)
    ex = jnp.exp(logits - m)
    s = jnp.sum(ex, axis=1, keepdims=True)
    probs = ex / s
    ii = lax.broadcasted_iota(jnp.int32, (N, E), 1)
    p1 = jnp.max(probs, axis=1, keepdims=True)
    a1 = jnp.min(jnp.where(probs == p1, ii, E), axis=1, keepdims=True)
    sel1 = ii == a1
    pm = jnp.where(sel1, -1.0, probs)
    p2 = jnp.max(pm, axis=1, keepdims=True)
    a2 = jnp.min(jnp.where(pm == p2, ii, E), axis=1, keepdims=True)
    denom = p1 + p2
    w1 = p1 / denom
    w2 = p2 / denom

    onehot = jnp.concatenate([sel1.astype(jnp.float32),
                              (ii == a2).astype(jnp.float32)], axis=0)

    li = lax.broadcasted_iota(jnp.int32, (512, 512), 0)
    lj = lax.broadcasted_iota(jnp.int32, (512, 512), 1)
    ltri = (li >= lj).astype(jnp.float32)

    carry = jnp.zeros((1, E), jnp.float32)
    cblocks = []
    for i in range(NS // 512):
        blk = lax.slice(onehot, (i * 512, 0), ((i + 1) * 512, E))
        c = lax.dot_general(ltri, blk, (((1,), (0,)), ((), ())),
                            preferred_element_type=jnp.float32) + carry
        cblocks.append(c)
        carry = carry + jnp.sum(blk, axis=0, keepdims=True)
    csum = jnp.concatenate(cblocks, axis=0)
    counts = carry
    mi = lax.broadcasted_iota(jnp.int32, (E, E), 0)
    mj = lax.broadcasted_iota(jnp.int32, (E, E), 1)
    mstrict = (mi < mj).astype(jnp.float32)
    offs = lax.dot_general(counts, mstrict, (((1,), (0,)), ((), ())),
                           preferred_element_type=jnp.float32,
                           precision=lax.Precision.HIGHEST)

    pos = jnp.sum(onehot * (csum - 1.0 + offs), axis=1,
                  keepdims=True)
    p_ref[...] = jnp.broadcast_to(pos, (NS, E))
    wcat = jnp.concatenate([w1, w2], axis=0)
    w_ref[...] = jnp.broadcast_to(wcat, (NS, E))
    offs_ref[...] = jnp.broadcast_to(offs, (E, E))


def _route(logits):
    return pl.pallas_call(
        _route_body,
        out_shape=[
            jax.ShapeDtypeStruct((NS, E), jnp.float32),
            jax.ShapeDtypeStruct((NS, E), jnp.float32),
            jax.ShapeDtypeStruct((E, E), jnp.float32),
        ],
    )(logits)


def _tile_maps(offs):
    starts = offs
    ends = jnp.concatenate([offs[1:], jnp.array([NS], jnp.int32)])
    rb = jnp.arange(RB, dtype=jnp.int32)
    lo = rb * BLK
    hi = lo + BLK
    e_lo = jnp.searchsorted(ends, lo, side="right").astype(jnp.int32)
    e_hi = (jnp.searchsorted(starts, hi, side="left") - 1).astype(jnp.int32)
    n = e_hi - e_lo + 1
    cumstart = jnp.concatenate(
        [jnp.zeros((1,), jnp.int32), jnp.cumsum(n)[:-1].astype(jnp.int32)])
    t = jnp.arange(T, dtype=jnp.int32)
    rb_t = jnp.clip(jnp.searchsorted(cumstart, t, side="right").astype(
        jnp.int32) - 1, 0, RB - 1)
    within = t - cumstart[rb_t]
    valid = (within < n[rb_t]).astype(jnp.int32)
    e_t = jnp.clip(e_lo[rb_t] + within, 0, E - 1)
    first = ((within == 0) & (valid == 1)).astype(jnp.int32)
    return rb_t, e_t, first, valid, starts, ends


def _sc_scatter_rows(x, p3):
    @functools.partial(
        pl.kernel,
        out_type=jax.ShapeDtypeStruct((NS, H), jnp.float32),
        mesh=_sc_mesh(),
        scratch_types=[
            pltpu.VMEM((NCH, CH), jnp.int32),
            pltpu.VMEM((CH, H), jnp.float32),
            pltpu.VMEM((CH, H), jnp.float32),
            pltpu.SemaphoreType.DMA,
            pltpu.SemaphoreType.DMA,
        ],
    )
    def k(x_hbm, p_hbm, xs_hbm, idx_v, rows0, rows1, semr, semw):
        wid = lax.axis_index("s") * 2 + lax.axis_index("c")
        pltpu.sync_copy(p_hbm.at[wid], idx_v)
        base = wid * SLOTS_W
        bufs = (rows0, rows1)
        rd = pltpu.async_copy(x_hbm.at[pl.ds(base % N, CH)], rows0, semr)
        scat = None
        for j in range(NCH):
            rd.wait()
            new_scat = pltpu.async_copy(bufs[j % 2], xs_hbm.at[idx_v.at[j]],
                                        semw)
            if j + 1 < NCH:
                if scat is not None:
                    scat.wait()
                row0 = (base + (j + 1) * CH) % N
                rd = pltpu.async_copy(x_hbm.at[pl.ds(row0, CH)],
                                      bufs[(j + 1) % 2], semr)
            scat = new_scat
        scat.wait()

    return k(x, p3)


def _sc_gather_rows(ys, p3):
    @functools.partial(
        pl.kernel,
        out_type=jax.ShapeDtypeStruct((NS, H), jnp.float32),
        mesh=_sc_mesh(),
        scratch_types=[
            pltpu.VMEM((NCH, CH), jnp.int32),
            pltpu.VMEM((CH, H), jnp.float32),
            pltpu.VMEM((CH, H), jnp.float32),
            pltpu.SemaphoreType.DMA,
            pltpu.SemaphoreType.DMA,
        ],
    )
    def k(ys_hbm, p_hbm, yo_hbm, idx_v, rows0, rows1, semr, semw):
        wid = lax.axis_index("s") * 2 + lax.axis_index("c")
        pltpu.sync_copy(p_hbm.at[wid], idx_v)
        base = wid * SLOTS_W
        bufs = (rows0, rows1)
        rd = pltpu.async_copy(ys_hbm.at[idx_v.at[0]], rows0, semr)
        wr = None
        for j in range(NCH):
            rd.wait()
            new_wr = pltpu.async_copy(bufs[j % 2],
                                      yo_hbm.at[pl.ds(base + j * CH, CH)],
                                      semw)
            if j + 1 < NCH:
                if wr is not None:
                    wr.wait()
                rd = pltpu.async_copy(ys_hbm.at[idx_v.at[j + 1]],
                                      bufs[(j + 1) % 2], semr)
            wr = new_wr
        wr.wait()

    return k(ys, p3)


def _gu_body(rbt_r, et_r, first_r, valid_r, st_r, en_r,
             xs_ref, wg_ref, wu_ref, h_ref):
    t = pl.program_id(1)
    e = et_r[t]
    rows = rbt_r[t] * BLK + lax.broadcasted_iota(jnp.int32, (BLK, 1), 0)
    m = ((rows >= st_r[e]) & (rows < en_r[e])).astype(jnp.float32)
    m = m * valid_r[t].astype(jnp.float32)
    x = xs_ref[...]
    gate = lax.dot_general(x, wg_ref[0], (((1,), (1,)), ((), ())),
                           preferred_element_type=jnp.float32)
    up = lax.dot_general(x, wu_ref[0], (((1,), (1,)), ((), ())),
                         preferred_element_type=jnp.float32)
    h = up * _gelu(gate) * m

    @pl.when(first_r[t] == 1)
    def _():
        h_ref[...] = h

    @pl.when(first_r[t] == 0)
    def _():
        h_ref[...] += h


def _gmm_gateup(rbt, et, first, valid, st, en, xs, w_gate_up):
    grid_spec = pltpu.PrefetchScalarGridSpec(
        num_scalar_prefetch=6,
        grid=(DB, T),
        in_specs=[
            pl.BlockSpec((BLK, H), lambda d, t, rbt, *_: (rbt[t], 0)),
            pl.BlockSpec((1, DBLK, H),
                         lambda d, t, rbt, et, *_: (et[t], d, 0)),
            pl.BlockSpec((1, DBLK, H),
                         lambda d, t, rbt, et, *_: (et[t], d + DB, 0)),
        ],
        out_specs=pl.BlockSpec((BLK, DBLK), lambda d, t, rbt, *_: (rbt[t], d)),
    )
    return pl.pallas_call(
        _gu_body,
        grid_spec=grid_spec,
        out_shape=jax.ShapeDtypeStruct((NS, DFF), jnp.float32),
        compiler_params=pltpu.CompilerParams(
            dimension_semantics=("arbitrary", "arbitrary")),
    )(rbt, et, first, valid, st, en, xs, w_gate_up, w_gate_up)


def _down_body(rbt_r, et_r, first_r, valid_r, st_r, en_r,
               h_ref, wd_ref, y_ref):
    t = pl.program_id(0)
    e = et_r[t]
    rows = rbt_r[t] * BLK + lax.broadcasted_iota(jnp.int32, (BLK, 1), 0)
    m = ((rows >= st_r[e]) & (rows < en_r[e])).astype(jnp.float32)
    m = m * valid_r[t].astype(jnp.float32)
    y = lax.dot_general(h_ref[...] * m, wd_ref[0], (((1,), (1,)), ((), ())),
                        preferred_element_type=jnp.float32)

    @pl.when(first_r[t] == 1)
    def _():
        y_ref[...] = y

    @pl.when(first_r[t] == 0)
    def _():
        y_ref[...] += y


def _gmm_down(rbt, et, first, valid, st, en, h, w_down):
    grid_spec = pltpu.PrefetchScalarGridSpec(
        num_scalar_prefetch=6,
        grid=(T,),
        in_specs=[
            pl.BlockSpec((BLK, DFF), lambda t, rbt, *_: (rbt[t], 0)),
            pl.BlockSpec((1, H, DFF), lambda t, rbt, et, *_: (et[t], 0, 0)),
        ],
        out_specs=pl.BlockSpec((BLK, H), lambda t, rbt, *_: (rbt[t], 0)),
    )
    return pl.pallas_call(
        _down_body,
        grid_spec=grid_spec,
        out_shape=jax.ShapeDtypeStruct((NS, H), jnp.float32),
        compiler_params=pltpu.CompilerParams(
            dimension_semantics=("arbitrary",)),
    )(rbt, et, first, valid, st, en, h, w_down)


def _comb_body(y1_ref, y2_ref, w1_ref, w2_ref, o_ref):
    o_ref[...] = (y1_ref[...] * w1_ref[:, 0:1] +
                  y2_ref[...] * w2_ref[:, 0:1])


def _combine(y_slot, w_b):
    blk = N // 8
    return pl.pallas_call(
        _comb_body,
        grid=(8,),
        in_specs=[
            pl.BlockSpec((blk, H), lambda i: (i, 0)),
            pl.BlockSpec((blk, H), lambda i: (i + 8, 0)),
            pl.BlockSpec((blk, E), lambda i: (i, 0)),
            pl.BlockSpec((blk, E), lambda i: (i + 8, 0)),
        ],
        out_specs=pl.BlockSpec((blk, H), lambda i: (i, 0)),
        out_shape=jax.ShapeDtypeStruct((N, H), jnp.float32),
    )(y_slot, y_slot, w_b, w_b)


def kernel(hidden_states, gate_w, w_gate_up, w_down):
    x = hidden_states.reshape(N, H)
    logits = _router_logits(x, gate_w)
    p_b, w_b, offs_b = _route(logits)
    p = p_b[:, 0].astype(jnp.int32)
    offs = offs_b[0, :].astype(jnp.int32)
    p3 = p.reshape(NW, NCH, CH)
    rbt, et, first, valid, st, en = _tile_maps(offs)
    xs = _sc_scatter_rows(x, p3)
    h = _gmm_gateup(rbt, et, first, valid, st, en, xs, w_gate_up)
    ys = _gmm_down(rbt, et, first, valid, st, en, h, w_down)
    y_slot = _sc_gather_rows(ys, p3)
    out = _combine(y_slot, w_b)
    return out.reshape(B, S, H)
